# baseline (speedup 1.0000x reference)
import functools

import jax
import jax.numpy as jnp
from jax import lax
from jax.experimental import pallas as pl
from jax.experimental.pallas import tpu as pltpu
from jax.experimental.pallas import tpu_sc as plsc

EMB = 32
G = 64
_NC = 2
_NS = 16
_NW = _NC * _NS
_CAP = 2048
_GC = 128
_NBLK = 32
_C3BLK = 10
_F32 = jnp.float32
_I32 = jnp.int32
_HI = lax.Precision.HIGHEST


def _leaky(x):
    return jnp.where(x >= 0, x, 0.01 * x)


def _full_spec(shape):
    return pl.BlockSpec(shape, lambda i: tuple(0 for _ in shape))


def _blk_spec(bn, minor):
    return pl.BlockSpec((bn, minor), lambda i: (i, 0))


def _sc_prepass(esrc, edst, npad):
    e = esrc.shape[0]
    echunk = e // _NW
    npn = npad // _NW
    nv = echunk // 16
    mesh = plsc.VectorSubcoreMesh(
        core_axis_name="c", subcore_axis_name="s",
        num_cores=_NC, num_subcores=_NS)

    @functools.partial(
        pl.kernel,
        out_type=[
            jax.ShapeDtypeStruct((_NW, _NW, _CAP), _I32),
            jax.ShapeDtypeStruct((_NW, _NW, _CAP), _I32),
            jax.ShapeDtypeStruct((_NW, _NW), _I32),
        ],
        mesh=mesh,
        scratch_types=[
            pltpu.VMEM((echunk,), _I32),
            pltpu.VMEM((echunk,), _I32),
            pltpu.VMEM((_CAP,), _I32),
            pltpu.VMEM((_CAP,), _I32),
            pltpu.VMEM((_NW,), _I32),
        ],
        compiler_params=pltpu.CompilerParams(needs_layout_passes=False),
    )
    def kern(esrc_h, edst_h, bsrc_h, bdst_h, cnt_h, src_v, dst_v, bs_v, bd_v, cnt_v):
        wid = lax.axis_index("s") * _NC + lax.axis_index("c")
        base = wid * echunk
        pltpu.sync_copy(esrc_h.at[pl.ds(base, echunk)], src_v)
        pltpu.sync_copy(edst_h.at[pl.ds(base, echunk)], dst_v)
        iota = lax.iota(_I32, 16)
        zeros16 = jnp.zeros((16,), _I32)
        dump16 = jnp.full((16,), npn, _I32)
        for o in range(_NW):
            lo = o * npn
            hi = lo + npn

            def vbody(k, cnt):
                idx = k * 16 + iota
                d = plsc.load_gather(dst_v, [idx])
                s = plsc.load_gather(src_v, [idx])
                m = (d >= lo) & (d < hi)
                plsc.store_compressed(bs_v.at[pl.ds(cnt, 16)], s, mask=m)
                plsc.store_compressed(bd_v.at[pl.ds(cnt, 16)], d - lo, mask=m)
                pc = plsc.all_reduce_population_count(m)[0]
                return jnp.minimum(cnt + pc, _CAP - 16)

            cnt = lax.fori_loop(0, nv, vbody, jnp.asarray(0, _I32))
            pend = ((cnt + 2 * _GC - 1) // (2 * _GC)) * (2 * _GC)
            for kk in range(2 * _GC // 16):
                pos = cnt + kk * 16 + iota
                mpad = pos < pend
                plsc.store_scatter(bs_v, [pos], zeros16, mask=mpad)
                plsc.store_scatter(bd_v, [pos], dump16, mask=mpad)
            pltpu.sync_copy(bs_v, bsrc_h.at[wid, o])
            pltpu.sync_copy(bd_v, bdst_h.at[wid, o])
            plsc.store_scatter(cnt_v, [jnp.full((16,), o, _I32)],
                               jnp.full((16,), cnt, _I32), mask=(iota == 0))
        pltpu.sync_copy(cnt_v, cnt_h.at[wid])

    return kern(esrc, edst)


def _sc_segmax(y, bsrc, bdst, cnts, npad):
    npn = npad // _NW
    mesh = plsc.VectorSubcoreMesh(
        core_axis_name="c", subcore_axis_name="s",
        num_cores=_NC, num_subcores=_NS)

    @functools.partial(
        pl.kernel,
        out_type=jax.ShapeDtypeStruct((npad * EMB,), _F32),
        mesh=mesh,
        scratch_types=[
            pltpu.VMEM(((npn + 1) * EMB,), _F32),
            pltpu.VMEM((_CAP,), _I32),
            pltpu.VMEM((_CAP,), _I32),
            pltpu.VMEM((_NW, _NW), _I32),
            pltpu.VMEM((_GC, EMB), _F32),
            pltpu.VMEM((_GC, EMB), _F32),
            pltpu.SemaphoreType.DMA,
            pltpu.SemaphoreType.DMA,
        ],
        compiler_params=pltpu.CompilerParams(
            needs_layout_passes=False, use_tc_tiling_on_sc=False),
    )
    def kern(y_h, bsrc_h, bdst_h, cnt_h, acc_h, acc_v, sb_v, db_v, cnt_v,
             msg_a, msg_b, sem_a, sem_b):
        wid = lax.axis_index("s") * _NC + lax.axis_index("c")
        o = wid
        iota = lax.iota(_I32, 16)
        neg = jnp.full((16,), -jnp.inf, _F32)

        def initb(r, _):
            acc_v[pl.ds(r * 16, 16)] = neg
            return 0

        lax.fori_loop(0, (npn + 1) * EMB // 16, initb, 0)
        pltpu.sync_copy(cnt_h, cnt_v)

        def _process(msg_v, cc):
            def gbody(k8, _3):
                dlv = db_v[pl.ds(cc * _GC + k8 * 16, 16)]
                for j2 in range(16):
                    base = dlv[j2] * EMB
                    jv = jnp.full((16,), k8 * 16 + j2, _I32)
                    a0 = acc_v[pl.ds(base, 16)]
                    a1 = acc_v[pl.ds(base + 16, 16)]
                    m0 = plsc.load_gather(msg_v, [jv, iota])
                    m1 = plsc.load_gather(msg_v, [jv, iota + 16])
                    acc_v[pl.ds(base, 16)] = jnp.maximum(a0, m0)
                    acc_v[pl.ds(base + 16, 16)] = jnp.maximum(a1, m1)
                return 0

            lax.fori_loop(0, _GC // 16, gbody, 0)

        def _gather(msg_v, sem, cc):
            return pltpu.async_copy(
                y_h.at[sb_v.at[pl.ds(cc * _GC, _GC)]], msg_v, sem)

        def wbody(w, _):
            wv = jnp.full((16,), w, _I32)
            ov = jnp.full((16,), o, _I32)
            cnt = plsc.load_gather(cnt_v, [wv, ov])[0]
            pltpu.sync_copy(bsrc_h.at[w, o], sb_v)
            pltpu.sync_copy(bdst_h.at[w, o], db_v)
            nch2 = (cnt + 2 * _GC - 1) // (2 * _GC)

            def cbody(cc, _2):
                _gather(msg_a, sem_a, cc).wait()
                _process(msg_a, cc)
                return 0

            lax.fori_loop(0, 2 * nch2, cbody, 0)
            return 0

        lax.fori_loop(0, _NW, wbody, 0)
        pltpu.sync_copy(acc_v.at[pl.ds(0, npn * EMB)],
                        acc_h.at[pl.ds(o * npn * EMB, npn * EMB)])

    return kern(y, bsrc, bdst, cnts).reshape(npad, EMB)


def _tc_emb(xr, We, be, Wm, bm):
    npad = xr.shape[0]
    bn = npad // _NBLK

    def body(xr_ref, We_ref, be_ref, Wm_ref, bm_ref, x_ref, y_ref):
        x = _leaky(xr_ref[...] * We_ref[...] + be_ref[...])
        x_ref[...] = x
        y_ref[...] = _leaky(jnp.dot(x, Wm_ref[...], precision=_HI) + bm_ref[...])

    return pl.pallas_call(
        body,
        grid=(_NBLK,),
        in_specs=[_blk_spec(bn, 1), _full_spec((1, EMB)), _full_spec((EMB,)),
                  _full_spec((EMB, EMB)), _full_spec((EMB,))],
        out_specs=[_blk_spec(bn, EMB), _blk_spec(bn, EMB)],
        out_shape=[jax.ShapeDtypeStruct((npad, EMB), _F32),
                   jax.ShapeDtypeStruct((npad, EMB), _F32)],
    )(xr, We, be, Wm, bm)


def _tc_b1(x, acc, bi, xg, agg_p, gate_p, feat_p, next_p, leaky_yn):
    npad = x.shape[0]
    bn = npad // _NBLK
    Wa, ba = agg_p
    Wg, bg = gate_p
    Wf, bf = feat_p
    Wn, bnn = next_p
    kn = Wn.shape[1]

    def body(x_ref, acc_ref, bi_ref, xg_ref, Wa_ref, ba_ref, Wg_ref, bg_ref,
             Wf_ref, bf_ref, Wn_ref, bn_ref,
             xn_ref, l_ref, feat_ref, yn_ref, m_ref, m2_ref, m_sc, m2_sc):
        i = pl.program_id(0)

        @pl.when(i == 0)
        def _():
            m_sc[...] = jnp.full((G,), -jnp.inf, _F32)
            m2_sc[...] = jnp.full((G,), -jnp.inf, _F32)

        x_ = x_ref[...]
        a = acc_ref[...]
        agg = jnp.where(jnp.isneginf(a), 0.0, a)
        bi_ = bi_ref[...]
        oh = (bi_ == lax.broadcasted_iota(_I32, (1, G), 1)).astype(_F32)
        xgb = jnp.dot(oh, xg_ref[...], precision=_HI)
        z = jnp.concatenate([x_, xgb, agg], axis=1)
        xn = _leaky(jnp.dot(z, Wa_ref[...], precision=_HI) + ba_ref[...]) + x_
        l = jnp.dot(xn, Wg_ref[...], precision=_HI) + bg_ref[...]
        feat = _leaky(jnp.dot(xn, Wf_ref[...], precision=_HI) + bf_ref[...])
        yn = jnp.dot(xn, Wn_ref[...], precision=_HI) + bn_ref[...]
        if leaky_yn:
            yn = _leaky(yn)
        m_sc[...] = jnp.maximum(m_sc[...], jnp.max(
            jnp.where(oh > 0, l, -jnp.inf), axis=0))
        m2_sc[...] = jnp.maximum(m2_sc[...], jnp.max(
            jnp.where(oh > 0, yn[:, 0:1], -jnp.inf), axis=0))
        xn_ref[...] = xn
        l_ref[...] = l
        feat_ref[...] = feat
        yn_ref[...] = yn
        m_ref[...] = m_sc[...]
        m2_ref[...] = m2_sc[...]

    return pl.pallas_call(
        body,
        grid=(_NBLK,),
        in_specs=[_blk_spec(bn, EMB), _blk_spec(bn, EMB), _blk_spec(bn, 1),
                  _full_spec((G, EMB)),
                  _full_spec((3 * EMB, EMB)), _full_spec((EMB,)),
                  _full_spec((EMB, 1)), _full_spec((1,)),
                  _full_spec((EMB, EMB)), _full_spec((EMB,)),
                  _full_spec((EMB, kn)), _full_spec((kn,))],
        out_specs=[_blk_spec(bn, EMB), _blk_spec(bn, 1), _blk_spec(bn, EMB),
                   _blk_spec(bn, kn), _full_spec((G,)), _full_spec((G,))],
        out_shape=[jax.ShapeDtypeStruct((npad, EMB), _F32),
                   jax.ShapeDtypeStruct((npad, 1), _F32),
                   jax.ShapeDtypeStruct((npad, EMB), _F32),
                   jax.ShapeDtypeStruct((npad, kn), _F32),
                   jax.ShapeDtypeStruct((G,), _F32),
                   jax.ShapeDtypeStruct((G,), _F32)],
        scratch_shapes=[pltpu.VMEM((G,), _F32), pltpu.VMEM((G,), _F32)],
    )(x, acc, bi, xg, Wa, ba, Wg, bg, Wf, bf, Wn, bnn)


def _tc_b2(l, feat, bi, m, xg, tr_p):
    npad = l.shape[0]
    bn = npad // _NBLK
    Wt, bt = tr_p

    def body(l_ref, feat_ref, bi_ref, m_ref, xg_ref, Wt_ref, bt_ref,
             xgn_ref, s_sc, a_sc):
        i = pl.program_id(0)

        @pl.when(i == 0)
        def _():
            s_sc[...] = jnp.zeros((G, 1), _F32)
            a_sc[...] = jnp.zeros((G, EMB), _F32)

        bi_ = bi_ref[...]
        oh = (bi_ == lax.broadcasted_iota(_I32, (1, G), 1)).astype(_F32)
        m_ = m_ref[...]
        mf = jnp.where(jnp.isneginf(m_), 0.0, m_)
        mg = jnp.dot(oh, mf[:, None], precision=_HI)
        e = jnp.exp(l_ref[...] - mg)
        dn = (((0,), (0,)), ((), ()))
        s_sc[...] += lax.dot_general(oh, e, dn, precision=_HI)
        a_sc[...] += lax.dot_general(oh, e * feat_ref[...], dn, precision=_HI)

        @pl.when(i == _NBLK - 1)
        def _():
            xga = a_sc[...] / (s_sc[...] + 1e-16)
            xg_ = xg_ref[...]
            cat = jnp.concatenate([xga, xg_], axis=1)
            xgn_ref[...] = _leaky(
                jnp.dot(cat, Wt_ref[...], precision=_HI) + bt_ref[...]) + xg_

    return pl.pallas_call(
        body,
        grid=(_NBLK,),
        in_specs=[_blk_spec(bn, 1), _blk_spec(bn, EMB), _blk_spec(bn, 1),
                  _full_spec((G,)), _full_spec((G, EMB)),
                  _full_spec((2 * EMB, EMB)), _full_spec((EMB,))],
        out_specs=[_full_spec((G, EMB))],
        out_shape=[jax.ShapeDtypeStruct((G, EMB), _F32)],
        scratch_shapes=[pltpu.VMEM((G, 1), _F32), pltpu.VMEM((G, EMB), _F32)],
    )(l, feat, bi, m, xg, Wt, bt)[0]


def _tc_c2(l1, bi, m1, n_real):
    npad = l1.shape[0]
    bn = npad // _NBLK

    def body(l1_ref, bi_ref, m1_ref, e1_ref, s1_ref, cnt_ref, st_ref, s_sc, c_sc):
        i = pl.program_id(0)

        @pl.when(i == 0)
        def _():
            s_sc[...] = jnp.zeros((G, 1), _F32)
            c_sc[...] = jnp.zeros((G, 1), _F32)

        bi_ = bi_ref[...]
        oh = (bi_ == lax.broadcasted_iota(_I32, (1, G), 1)).astype(_F32)
        m_ = m1_ref[...]
        mf = jnp.where(jnp.isneginf(m_), 0.0, m_)
        mg = jnp.dot(oh, mf[:, None], precision=_HI)
        rowid = i * bn + lax.broadcasted_iota(_I32, (bn, 1), 0)
        e1 = jnp.where(rowid < n_real, jnp.exp(l1_ref[...] - mg), 0.0)
        dn = (((0,), (0,)), ((), ()))
        s_sc[...] += lax.dot_general(oh, e1, dn, precision=_HI)
        c_sc[...] += jnp.sum(oh, axis=0)[:, None]
        e1_ref[...] = e1

        @pl.when(i == _NBLK - 1)
        def _():
            s1_ref[...] = s_sc[...][:, 0]
            cnts = c_sc[...][:, 0]
            cnt_ref[...] = cnts.astype(_I32)
            rr = lax.broadcasted_iota(_I32, (G, G), 0)
            cc = lax.broadcasted_iota(_I32, (G, G), 1)
            lt = (cc < rr).astype(_F32)
            st_ref[...] = jnp.dot(lt, cnts[:, None], precision=_HI)[:, 0].astype(_I32)

    return pl.pallas_call(
        body,
        grid=(_NBLK,),
        in_specs=[_blk_spec(bn, 1), _blk_spec(bn, 1), _full_spec((G,))],
        out_specs=[_blk_spec(bn, 1), _full_spec((G,)), _full_spec((G,)),
                   _full_spec((G,))],
        out_shape=[jax.ShapeDtypeStruct((npad, 1), _F32),
                   jax.ShapeDtypeStruct((G,), _F32),
                   jax.ShapeDtypeStruct((G,), _I32),
                   jax.ShapeDtypeStruct((G,), _I32)],
        scratch_shapes=[pltpu.VMEM((G, 1), _F32), pltpu.VMEM((G, 1), _F32)],
    )(l1, bi, m1)


def _tc_c2b(e1, bi, s1):
    npad = e1.shape[0]
    bn = npad // _NBLK

    def body(e1_ref, bi_ref, s1_ref, np_ref):
        bi_ = bi_ref[...]
        oh = (bi_ == lax.broadcasted_iota(_I32, (1, G), 1)).astype(_F32)
        s1g = jnp.dot(oh, s1_ref[...][:, None], precision=_HI)
        np_ref[...] = e1_ref[...] / (s1g + 1e-16)

    return pl.pallas_call(
        body,
        grid=(_NBLK,),
        in_specs=[_blk_spec(bn, 1), _blk_spec(bn, 1), _full_spec((G,))],
        out_specs=[_blk_spec(bn, 1)],
        out_shape=[jax.ShapeDtypeStruct((npad, 1), _F32)],
    )(e1, bi, s1)[0]


def _tc_c3(np128):
    nrows = np128.shape[0]
    br = nrows // _C3BLK

    def body(np_ref, c_ref, carry_sc):
        i = pl.program_id(0)

        @pl.when(i == 0)
        def _():
            carry_sc[0] = 0.0

        v = np_ref[...]
        rr = lax.broadcasted_iota(_I32, (128, 128), 0)
        cc = lax.broadcasted_iota(_I32, (128, 128), 1)
        t = (rr <= cc).astype(_F32)
        rowcs = jnp.dot(v, t, precision=_HI)
        rowsum = rowcs[:, 127:128]
        r2 = lax.broadcasted_iota(_I32, (br, br), 0)
        c2 = lax.broadcasted_iota(_I32, (br, br), 1)
        lt = (c2 < r2).astype(_F32)
        rpref = jnp.dot(lt, rowsum, precision=_HI)
        carry = carry_sc[0]
        c_ref[...] = rowcs + rpref + carry
        carry_sc[0] = carry + (rpref[br - 1, 0] + rowsum[br - 1, 0])

    return pl.pallas_call(
        body,
        grid=(_C3BLK,),
        in_specs=[_blk_spec(br, 128)],
        out_specs=[_blk_spec(br, 128)],
        out_shape=[jax.ShapeDtypeStruct((nrows, 128), _F32)],
        scratch_shapes=[pltpu.SMEM((1,), _F32)],
    )(np128)[0]


def _tc_c3b(c_col, bi, starts, cnts, u1):
    npad = c_col.shape[0]
    bn = npad // _NBLK

    def body(c_ref, bi_ref, st_ref, cnt_ref, u1_ref, a1_ref, off_sc, k_sc):
        i = pl.program_id(0)

        @pl.when(i == 0)
        def _():
            off_sc[...] = jnp.zeros((G, 1), _F32)
            k_sc[...] = jnp.zeros((G, 1), _F32)

        bi_ = bi_ref[...]
        oh = (bi_ == lax.broadcasted_iota(_I32, (1, G), 1)).astype(_F32)
        c = c_ref[...]
        gpos = i * bn + lax.broadcasted_iota(_I32, (bn, 1), 0)
        st = st_ref[...]
        pick = (gpos == (st[None, :] - 1)).astype(_F32)
        dn = (((0,), (0,)), ((), ()))
        off_sc[...] += lax.dot_general(pick, c, dn, precision=_HI)
        offg = jnp.dot(oh, off_sc[...], precision=_HI)
        u1g = jnp.dot(oh, u1_ref[...][:, None], precision=_HI)
        kc = ((c - offg) < u1g).astype(_F32)
        k_sc[...] += lax.dot_general(oh, kc, dn, precision=_HI)

        @pl.when(i == _NBLK - 1)
        def _():
            k = k_sc[...][:, 0].astype(_I32)
            a1_ref[...] = jnp.clip(k, 0, jnp.maximum(cnt_ref[...] - 1, 0))

    return pl.pallas_call(
        body,
        grid=(_NBLK,),
        in_specs=[_blk_spec(bn, 1), _blk_spec(bn, 1), _full_spec((G,)),
                  _full_spec((G,)), _full_spec((G,))],
        out_specs=[_full_spec((G,))],
        out_shape=[jax.ShapeDtypeStruct((G,), _I32)],
        scratch_shapes=[pltpu.VMEM((G, 1), _F32), pltpu.VMEM((G, 1), _F32)],
    )(c_col, bi, starts, cnts, u1)[0]


def _tc_c4(np_col, starts, a1, xg, u0, v_p, a0_p):
    npad = np_col.shape[0]
    bn = npad // _NBLK
    Wv, bv = v_p
    Wa0, ba0 = a0_p

    def body(np_ref, st_ref, a1_ref, xg_ref, u0_ref, Wv_ref, bv_ref,
             Wa0_ref, ba0_ref, val_ref, af_ref, a0_ref, p_sc):
        i = pl.program_id(0)

        @pl.when(i == 0)
        def _():
            p_sc[...] = jnp.zeros((G, 1), _F32)

        gpos = i * bn + lax.broadcasted_iota(_I32, (bn, 1), 0)
        t = st_ref[...] + a1_ref[...]
        pick = (gpos == t[None, :]).astype(_F32)
        dn = (((0,), (0,)), ((), ()))
        p_sc[...] += lax.dot_general(pick, np_ref[...], dn, precision=_HI)

        @pl.when(i == _NBLK - 1)
        def _():
            xg_ = xg_ref[...]
            val_ref[...] = jnp.dot(xg_, Wv_ref[...], precision=_HI) + bv_ref[...]
            tt = jnp.dot(xg_, Wa0_ref[...], precision=_HI) + ba0_ref[...]
            mm = jnp.max(tt, axis=1, keepdims=True)
            ex = jnp.exp(tt - mm)
            p0 = ex / jnp.sum(ex, axis=1, keepdims=True)
            a0 = (u0_ref[...] >= p0[:, 0]).astype(_I32)
            a0_ref[...] = a0
            af_ref[...] = jnp.where(a0 == 1, p0[:, 1], p0[:, 0] * p_sc[...][:, 0])

    return pl.pallas_call(
        body,
        grid=(_NBLK,),
        in_specs=[_blk_spec(bn, 1), _full_spec((G,)), _full_spec((G,)),
                  _full_spec((G, EMB)), _full_spec((G,)),
                  _full_spec((EMB, 1)), _full_spec((1,)),
                  _full_spec((EMB, 2)), _full_spec((2,))],
        out_specs=[_full_spec((G, 1)), _full_spec((G,)), _full_spec((G,))],
        out_shape=[jax.ShapeDtypeStruct((G, 1), _F32),
                   jax.ShapeDtypeStruct((G,), _F32),
                   jax.ShapeDtypeStruct((G,), _I32)],
        scratch_shapes=[pltpu.VMEM((G, 1), _F32)],
    )(np_col, starts, a1, xg, u0, Wv, bv, Wa0, ba0)


def kernel(x_raw, edge_index, batch_ind, u0, u1, params):
    n = x_raw.shape[0]
    g = u0.shape[0]
    unit = 128 * 8 * _C3BLK
    npad = ((n + unit - 1) // unit) * unit
    while npad % _NW or (npad // _NBLK) % 8 or (npad // 128 // _C3BLK) % 8:
        npad += unit

    xr = jnp.concatenate([x_raw, jnp.zeros((npad - n, 1), _F32)], axis=0)
    bi = jnp.concatenate(
        [batch_ind, jnp.full((npad - n,), g, _I32)], axis=0).reshape(npad, 1)
    esrc = edge_index[0]
    edst = edge_index[1]

    We, be = params["emb"]
    steps = params["steps"]
    x, y = _tc_emb(xr, We, be, steps[0]["mess"][0], steps[0]["mess"][1])
    bsrc, bdst, cnts = _sc_prepass(esrc, edst, npad)

    xg = jnp.zeros((g, EMB), _F32)
    m1 = None
    for s in range(3):
        p = steps[s]
        acc = _sc_segmax(y, bsrc, bdst, cnts, npad)
        if s < 2:
            next_p = steps[s + 1]["mess"]
            leaky_yn = True
        else:
            next_p = params["a1"]
            leaky_yn = False
        x, l, feat, y, m, m2 = _tc_b1(
            x, acc, bi, xg, p["agg"], p["gate"], p["feat"], next_p, leaky_yn)
        xg = _tc_b2(l, feat, bi, m, xg, p["tr"])
        m1 = m2

    e1, s1, cntg, starts = _tc_c2(y, bi, m1, n)
    np_col = _tc_c2b(e1, bi, s1)
    c128 = _tc_c3(np_col.reshape(npad // 128, 128))
    c_col = c128.reshape(npad, 1)
    a1 = _tc_c3b(c_col, bi, starts, cntg, u1)
    value, af, a0 = _tc_c4(np_col, starts, a1, xg, u0, params["v"], params["a0"])
    return (value, af, np_col[:n, 0], a0, a1)


# speedup vs baseline: 1.0013x; 1.0013x over previous
import functools

import jax
import jax.numpy as jnp
from jax import lax
from jax.experimental import pallas as pl
from jax.experimental.pallas import tpu as pltpu
from jax.experimental.pallas import tpu_sc as plsc

EMB = 32
G = 64
_NC = 2
_NS = 16
_NW = _NC * _NS
_CAP = 2048
_GC = 128
_NBLK = 32
_C3BLK = 10
_F32 = jnp.float32
_I32 = jnp.int32
_HI = lax.Precision.HIGHEST


def _leaky(x):
    return jnp.where(x >= 0, x, 0.01 * x)


def _full_spec(shape):
    return pl.BlockSpec(shape, lambda i: tuple(0 for _ in shape))


def _blk_spec(bn, minor):
    return pl.BlockSpec((bn, minor), lambda i: (i, 0))


def _sc_prepass(esrc, edst, npad):
    e = esrc.shape[0]
    echunk = e // _NW
    npn = npad // _NW
    nv = echunk // 16
    mesh = plsc.VectorSubcoreMesh(
        core_axis_name="c", subcore_axis_name="s",
        num_cores=_NC, num_subcores=_NS)

    @functools.partial(
        pl.kernel,
        out_type=[
            jax.ShapeDtypeStruct((_NW, _NW, _CAP), _I32),
            jax.ShapeDtypeStruct((_NW, _NW, _CAP), _I32),
            jax.ShapeDtypeStruct((_NW, _NW), _I32),
        ],
        mesh=mesh,
        scratch_types=[
            pltpu.VMEM((echunk,), _I32),
            pltpu.VMEM((echunk,), _I32),
            pltpu.VMEM((_CAP,), _I32),
            pltpu.VMEM((_CAP,), _I32),
            pltpu.VMEM((_NW,), _I32),
        ],
        compiler_params=pltpu.CompilerParams(needs_layout_passes=False),
    )
    def kern(esrc_h, edst_h, bsrc_h, bdst_h, cnt_h, src_v, dst_v, bs_v, bd_v, cnt_v):
        wid = lax.axis_index("s") * _NC + lax.axis_index("c")
        base = wid * echunk
        pltpu.sync_copy(esrc_h.at[pl.ds(base, echunk)], src_v)
        pltpu.sync_copy(edst_h.at[pl.ds(base, echunk)], dst_v)
        iota = lax.iota(_I32, 16)
        zeros16 = jnp.zeros((16,), _I32)
        dump16 = jnp.full((16,), npn, _I32)
        for o in range(_NW):
            lo = o * npn
            hi = lo + npn

            def vbody(k, cnt):
                idx = k * 16 + iota
                d = plsc.load_gather(dst_v, [idx])
                s = plsc.load_gather(src_v, [idx])
                m = (d >= lo) & (d < hi)
                plsc.store_compressed(bs_v.at[pl.ds(cnt, 16)], s, mask=m)
                plsc.store_compressed(bd_v.at[pl.ds(cnt, 16)], d - lo, mask=m)
                pc = plsc.all_reduce_population_count(m)[0]
                return jnp.minimum(cnt + pc, _CAP - 16)

            cnt = lax.fori_loop(0, nv, vbody, jnp.asarray(0, _I32))
            pend = ((cnt + 2 * _GC - 1) // (2 * _GC)) * (2 * _GC)
            for kk in range(2 * _GC // 16):
                pos = cnt + kk * 16 + iota
                mpad = pos < pend
                plsc.store_scatter(bs_v, [pos], zeros16, mask=mpad)
                plsc.store_scatter(bd_v, [pos], dump16, mask=mpad)
            pltpu.sync_copy(bs_v, bsrc_h.at[wid, o])
            pltpu.sync_copy(bd_v, bdst_h.at[wid, o])
            plsc.store_scatter(cnt_v, [jnp.full((16,), o, _I32)],
                               jnp.full((16,), cnt, _I32), mask=(iota == 0))
        pltpu.sync_copy(cnt_v, cnt_h.at[wid])

    return kern(esrc, edst)


def _sc_segmax(y, bsrc, bdst, cnts, npad):
    npn = npad // _NW
    mesh = plsc.VectorSubcoreMesh(
        core_axis_name="c", subcore_axis_name="s",
        num_cores=_NC, num_subcores=_NS)

    @functools.partial(
        pl.kernel,
        out_type=jax.ShapeDtypeStruct((npad * EMB,), _F32),
        mesh=mesh,
        scratch_types=[
            pltpu.VMEM(((npn + 1) * EMB,), _F32),
            pltpu.VMEM((_CAP,), _I32),
            pltpu.VMEM((_CAP,), _I32),
            pltpu.VMEM((_NW, _NW), _I32),
            pltpu.VMEM((_GC, EMB), _F32),
            pltpu.VMEM((_GC, EMB), _F32),
            pltpu.SemaphoreType.DMA,
            pltpu.SemaphoreType.DMA,
        ],
        compiler_params=pltpu.CompilerParams(
            needs_layout_passes=False, use_tc_tiling_on_sc=False),
    )
    def kern(y_h, bsrc_h, bdst_h, cnt_h, acc_h, acc_v, sb_v, db_v, cnt_v,
             msg_a, msg_b, sem_a, sem_b):
        wid = lax.axis_index("s") * _NC + lax.axis_index("c")
        o = wid
        iota = lax.iota(_I32, 16)
        neg = jnp.full((16,), -jnp.inf, _F32)

        def initb(r, _):
            acc_v[pl.ds(r * 16, 16)] = neg
            return 0

        lax.fori_loop(0, (npn + 1) * EMB // 16, initb, 0)
        pltpu.sync_copy(cnt_h, cnt_v)

        def _process(msg_v, cc):
            def gbody(k8, _3):
                dlv = db_v[pl.ds(cc * _GC + k8 * 16, 16)]
                basev = dlv * EMB
                for j2 in range(16):
                    jv = jnp.full((16,), k8 * 16 + j2, _I32)
                    i0 = jnp.full((16,), basev[j2], _I32) + iota
                    i1 = i0 + 16
                    a0 = plsc.load_gather(acc_v, [i0])
                    a1 = plsc.load_gather(acc_v, [i1])
                    m0 = plsc.load_gather(msg_v, [jv, iota])
                    m1 = plsc.load_gather(msg_v, [jv, iota + 16])
                    plsc.store_scatter(acc_v, [i0], jnp.maximum(a0, m0))
                    plsc.store_scatter(acc_v, [i1], jnp.maximum(a1, m1))
                return 0

            lax.fori_loop(0, _GC // 16, gbody, 0)

        def _gather(msg_v, sem, cc):
            return pltpu.async_copy(
                y_h.at[sb_v.at[pl.ds(cc * _GC, _GC)]], msg_v, sem)

        def wbody(w, _):
            wv = jnp.full((16,), w, _I32)
            ov = jnp.full((16,), o, _I32)
            cnt = plsc.load_gather(cnt_v, [wv, ov])[0]
            pltpu.sync_copy(bsrc_h.at[w, o], sb_v)
            pltpu.sync_copy(bdst_h.at[w, o], db_v)
            nch2 = (cnt + 2 * _GC - 1) // (2 * _GC)

            def cbody(cc, _2):
                _gather(msg_a, sem_a, cc).wait()
                _process(msg_a, cc)
                return 0

            lax.fori_loop(0, 2 * nch2, cbody, 0)
            return 0

        lax.fori_loop(0, _NW, wbody, 0)
        pltpu.sync_copy(acc_v.at[pl.ds(0, npn * EMB)],
                        acc_h.at[pl.ds(o * npn * EMB, npn * EMB)])

    return kern(y, bsrc, bdst, cnts).reshape(npad, EMB)


def _tc_emb(xr, We, be, Wm, bm):
    npad = xr.shape[0]
    bn = npad // _NBLK

    def body(xr_ref, We_ref, be_ref, Wm_ref, bm_ref, x_ref, y_ref):
        x = _leaky(xr_ref[...] * We_ref[...] + be_ref[...])
        x_ref[...] = x
        y_ref[...] = _leaky(jnp.dot(x, Wm_ref[...], precision=_HI) + bm_ref[...])

    return pl.pallas_call(
        body,
        grid=(_NBLK,),
        in_specs=[_blk_spec(bn, 1), _full_spec((1, EMB)), _full_spec((EMB,)),
                  _full_spec((EMB, EMB)), _full_spec((EMB,))],
        out_specs=[_blk_spec(bn, EMB), _blk_spec(bn, EMB)],
        out_shape=[jax.ShapeDtypeStruct((npad, EMB), _F32),
                   jax.ShapeDtypeStruct((npad, EMB), _F32)],
    )(xr, We, be, Wm, bm)


def _tc_b1(x, acc, bi, xg, agg_p, gate_p, feat_p, next_p, leaky_yn):
    npad = x.shape[0]
    bn = npad // _NBLK
    Wa, ba = agg_p
    Wg, bg = gate_p
    Wf, bf = feat_p
    Wn, bnn = next_p
    kn = Wn.shape[1]

    def body(x_ref, acc_ref, bi_ref, xg_ref, Wa_ref, ba_ref, Wg_ref, bg_ref,
             Wf_ref, bf_ref, Wn_ref, bn_ref,
             xn_ref, l_ref, feat_ref, yn_ref, m_ref, m2_ref, m_sc, m2_sc):
        i = pl.program_id(0)

        @pl.when(i == 0)
        def _():
            m_sc[...] = jnp.full((G,), -jnp.inf, _F32)
            m2_sc[...] = jnp.full((G,), -jnp.inf, _F32)

        x_ = x_ref[...]
        a = acc_ref[...]
        agg = jnp.where(jnp.isneginf(a), 0.0, a)
        bi_ = bi_ref[...]
        oh = (bi_ == lax.broadcasted_iota(_I32, (1, G), 1)).astype(_F32)
        xgb = jnp.dot(oh, xg_ref[...], precision=_HI)
        z = jnp.concatenate([x_, xgb, agg], axis=1)
        xn = _leaky(jnp.dot(z, Wa_ref[...], precision=_HI) + ba_ref[...]) + x_
        l = jnp.dot(xn, Wg_ref[...], precision=_HI) + bg_ref[...]
        feat = _leaky(jnp.dot(xn, Wf_ref[...], precision=_HI) + bf_ref[...])
        yn = jnp.dot(xn, Wn_ref[...], precision=_HI) + bn_ref[...]
        if leaky_yn:
            yn = _leaky(yn)
        m_sc[...] = jnp.maximum(m_sc[...], jnp.max(
            jnp.where(oh > 0, l, -jnp.inf), axis=0))
        m2_sc[...] = jnp.maximum(m2_sc[...], jnp.max(
            jnp.where(oh > 0, yn[:, 0:1], -jnp.inf), axis=0))
        xn_ref[...] = xn
        l_ref[...] = l
        feat_ref[...] = feat
        yn_ref[...] = yn
        m_ref[...] = m_sc[...]
        m2_ref[...] = m2_sc[...]

    return pl.pallas_call(
        body,
        grid=(_NBLK,),
        in_specs=[_blk_spec(bn, EMB), _blk_spec(bn, EMB), _blk_spec(bn, 1),
                  _full_spec((G, EMB)),
                  _full_spec((3 * EMB, EMB)), _full_spec((EMB,)),
                  _full_spec((EMB, 1)), _full_spec((1,)),
                  _full_spec((EMB, EMB)), _full_spec((EMB,)),
                  _full_spec((EMB, kn)), _full_spec((kn,))],
        out_specs=[_blk_spec(bn, EMB), _blk_spec(bn, 1), _blk_spec(bn, EMB),
                   _blk_spec(bn, kn), _full_spec((G,)), _full_spec((G,))],
        out_shape=[jax.ShapeDtypeStruct((npad, EMB), _F32),
                   jax.ShapeDtypeStruct((npad, 1), _F32),
                   jax.ShapeDtypeStruct((npad, EMB), _F32),
                   jax.ShapeDtypeStruct((npad, kn), _F32),
                   jax.ShapeDtypeStruct((G,), _F32),
                   jax.ShapeDtypeStruct((G,), _F32)],
        scratch_shapes=[pltpu.VMEM((G,), _F32), pltpu.VMEM((G,), _F32)],
    )(x, acc, bi, xg, Wa, ba, Wg, bg, Wf, bf, Wn, bnn)


def _tc_b2(l, feat, bi, m, xg, tr_p):
    npad = l.shape[0]
    bn = npad // _NBLK
    Wt, bt = tr_p

    def body(l_ref, feat_ref, bi_ref, m_ref, xg_ref, Wt_ref, bt_ref,
             xgn_ref, s_sc, a_sc):
        i = pl.program_id(0)

        @pl.when(i == 0)
        def _():
            s_sc[...] = jnp.zeros((G, 1), _F32)
            a_sc[...] = jnp.zeros((G, EMB), _F32)

        bi_ = bi_ref[...]
        oh = (bi_ == lax.broadcasted_iota(_I32, (1, G), 1)).astype(_F32)
        m_ = m_ref[...]
        mf = jnp.where(jnp.isneginf(m_), 0.0, m_)
        mg = jnp.dot(oh, mf[:, None], precision=_HI)
        e = jnp.exp(l_ref[...] - mg)
        dn = (((0,), (0,)), ((), ()))
        s_sc[...] += lax.dot_general(oh, e, dn, precision=_HI)
        a_sc[...] += lax.dot_general(oh, e * feat_ref[...], dn, precision=_HI)

        @pl.when(i == _NBLK - 1)
        def _():
            xga = a_sc[...] / (s_sc[...] + 1e-16)
            xg_ = xg_ref[...]
            cat = jnp.concatenate([xga, xg_], axis=1)
            xgn_ref[...] = _leaky(
                jnp.dot(cat, Wt_ref[...], precision=_HI) + bt_ref[...]) + xg_

    return pl.pallas_call(
        body,
        grid=(_NBLK,),
        in_specs=[_blk_spec(bn, 1), _blk_spec(bn, EMB), _blk_spec(bn, 1),
                  _full_spec((G,)), _full_spec((G, EMB)),
                  _full_spec((2 * EMB, EMB)), _full_spec((EMB,))],
        out_specs=[_full_spec((G, EMB))],
        out_shape=[jax.ShapeDtypeStruct((G, EMB), _F32)],
        scratch_shapes=[pltpu.VMEM((G, 1), _F32), pltpu.VMEM((G, EMB), _F32)],
    )(l, feat, bi, m, xg, Wt, bt)[0]


def _tc_c2(l1, bi, m1, n_real):
    npad = l1.shape[0]
    bn = npad // _NBLK

    def body(l1_ref, bi_ref, m1_ref, e1_ref, s1_ref, cnt_ref, st_ref, s_sc, c_sc):
        i = pl.program_id(0)

        @pl.when(i == 0)
        def _():
            s_sc[...] = jnp.zeros((G, 1), _F32)
            c_sc[...] = jnp.zeros((G, 1), _F32)

        bi_ = bi_ref[...]
        oh = (bi_ == lax.broadcasted_iota(_I32, (1, G), 1)).astype(_F32)
        m_ = m1_ref[...]
        mf = jnp.where(jnp.isneginf(m_), 0.0, m_)
        mg = jnp.dot(oh, mf[:, None], precision=_HI)
        rowid = i * bn + lax.broadcasted_iota(_I32, (bn, 1), 0)
        e1 = jnp.where(rowid < n_real, jnp.exp(l1_ref[...] - mg), 0.0)
        dn = (((0,), (0,)), ((), ()))
        s_sc[...] += lax.dot_general(oh, e1, dn, precision=_HI)
        c_sc[...] += jnp.sum(oh, axis=0)[:, None]
        e1_ref[...] = e1

        @pl.when(i == _NBLK - 1)
        def _():
            s1_ref[...] = s_sc[...][:, 0]
            cnts = c_sc[...][:, 0]
            cnt_ref[...] = cnts.astype(_I32)
            rr = lax.broadcasted_iota(_I32, (G, G), 0)
            cc = lax.broadcasted_iota(_I32, (G, G), 1)
            lt = (cc < rr).astype(_F32)
            st_ref[...] = jnp.dot(lt, cnts[:, None], precision=_HI)[:, 0].astype(_I32)

    return pl.pallas_call(
        body,
        grid=(_NBLK,),
        in_specs=[_blk_spec(bn, 1), _blk_spec(bn, 1), _full_spec((G,))],
        out_specs=[_blk_spec(bn, 1), _full_spec((G,)), _full_spec((G,)),
                   _full_spec((G,))],
        out_shape=[jax.ShapeDtypeStruct((npad, 1), _F32),
                   jax.ShapeDtypeStruct((G,), _F32),
                   jax.ShapeDtypeStruct((G,), _I32),
                   jax.ShapeDtypeStruct((G,), _I32)],
        scratch_shapes=[pltpu.VMEM((G, 1), _F32), pltpu.VMEM((G, 1), _F32)],
    )(l1, bi, m1)


def _tc_c2b(e1, bi, s1):
    npad = e1.shape[0]
    bn = npad // _NBLK

    def body(e1_ref, bi_ref, s1_ref, np_ref):
        bi_ = bi_ref[...]
        oh = (bi_ == lax.broadcasted_iota(_I32, (1, G), 1)).astype(_F32)
        s1g = jnp.dot(oh, s1_ref[...][:, None], precision=_HI)
        np_ref[...] = e1_ref[...] / (s1g + 1e-16)

    return pl.pallas_call(
        body,
        grid=(_NBLK,),
        in_specs=[_blk_spec(bn, 1), _blk_spec(bn, 1), _full_spec((G,))],
        out_specs=[_blk_spec(bn, 1)],
        out_shape=[jax.ShapeDtypeStruct((npad, 1), _F32)],
    )(e1, bi, s1)[0]


def _tc_c3(np128):
    nrows = np128.shape[0]
    br = nrows // _C3BLK

    def body(np_ref, c_ref, carry_sc):
        i = pl.program_id(0)

        @pl.when(i == 0)
        def _():
            carry_sc[0] = 0.0

        v = np_ref[...]
        rr = lax.broadcasted_iota(_I32, (128, 128), 0)
        cc = lax.broadcasted_iota(_I32, (128, 128), 1)
        t = (rr <= cc).astype(_F32)
        rowcs = jnp.dot(v, t, precision=_HI)
        rowsum = rowcs[:, 127:128]
        r2 = lax.broadcasted_iota(_I32, (br, br), 0)
        c2 = lax.broadcasted_iota(_I32, (br, br), 1)
        lt = (c2 < r2).astype(_F32)
        rpref = jnp.dot(lt, rowsum, precision=_HI)
        carry = carry_sc[0]
        c_ref[...] = rowcs + rpref + carry
        carry_sc[0] = carry + (rpref[br - 1, 0] + rowsum[br - 1, 0])

    return pl.pallas_call(
        body,
        grid=(_C3BLK,),
        in_specs=[_blk_spec(br, 128)],
        out_specs=[_blk_spec(br, 128)],
        out_shape=[jax.ShapeDtypeStruct((nrows, 128), _F32)],
        scratch_shapes=[pltpu.SMEM((1,), _F32)],
    )(np128)[0]


def _tc_c3b(c_col, bi, starts, cnts, u1):
    npad = c_col.shape[0]
    bn = npad // _NBLK

    def body(c_ref, bi_ref, st_ref, cnt_ref, u1_ref, a1_ref, off_sc, k_sc):
        i = pl.program_id(0)

        @pl.when(i == 0)
        def _():
            off_sc[...] = jnp.zeros((G, 1), _F32)
            k_sc[...] = jnp.zeros((G, 1), _F32)

        bi_ = bi_ref[...]
        oh = (bi_ == lax.broadcasted_iota(_I32, (1, G), 1)).astype(_F32)
        c = c_ref[...]
        gpos = i * bn + lax.broadcasted_iota(_I32, (bn, 1), 0)
        st = st_ref[...]
        pick = (gpos == (st[None, :] - 1)).astype(_F32)
        dn = (((0,), (0,)), ((), ()))
        off_sc[...] += lax.dot_general(pick, c, dn, precision=_HI)
        offg = jnp.dot(oh, off_sc[...], precision=_HI)
        u1g = jnp.dot(oh, u1_ref[...][:, None], precision=_HI)
        kc = ((c - offg) < u1g).astype(_F32)
        k_sc[...] += lax.dot_general(oh, kc, dn, precision=_HI)

        @pl.when(i == _NBLK - 1)
        def _():
            k = k_sc[...][:, 0].astype(_I32)
            a1_ref[...] = jnp.clip(k, 0, jnp.maximum(cnt_ref[...] - 1, 0))

    return pl.pallas_call(
        body,
        grid=(_NBLK,),
        in_specs=[_blk_spec(bn, 1), _blk_spec(bn, 1), _full_spec((G,)),
                  _full_spec((G,)), _full_spec((G,))],
        out_specs=[_full_spec((G,))],
        out_shape=[jax.ShapeDtypeStruct((G,), _I32)],
        scratch_shapes=[pltpu.VMEM((G, 1), _F32), pltpu.VMEM((G, 1), _F32)],
    )(c_col, bi, starts, cnts, u1)[0]


def _tc_c4(np_col, starts, a1, xg, u0, v_p, a0_p):
    npad = np_col.shape[0]
    bn = npad // _NBLK
    Wv, bv = v_p
    Wa0, ba0 = a0_p

    def body(np_ref, st_ref, a1_ref, xg_ref, u0_ref, Wv_ref, bv_ref,
             Wa0_ref, ba0_ref, val_ref, af_ref, a0_ref, p_sc):
        i = pl.program_id(0)

        @pl.when(i == 0)
        def _():
            p_sc[...] = jnp.zeros((G, 1), _F32)

        gpos = i * bn + lax.broadcasted_iota(_I32, (bn, 1), 0)
        t = st_ref[...] + a1_ref[...]
        pick = (gpos == t[None, :]).astype(_F32)
        dn = (((0,), (0,)), ((), ()))
        p_sc[...] += lax.dot_general(pick, np_ref[...], dn, precision=_HI)

        @pl.when(i == _NBLK - 1)
        def _():
            xg_ = xg_ref[...]
            val_ref[...] = jnp.dot(xg_, Wv_ref[...], precision=_HI) + bv_ref[...]
            tt = jnp.dot(xg_, Wa0_ref[...], precision=_HI) + ba0_ref[...]
            mm = jnp.max(tt, axis=1, keepdims=True)
            ex = jnp.exp(tt - mm)
            p0 = ex / jnp.sum(ex, axis=1, keepdims=True)
            a0 = (u0_ref[...] >= p0[:, 0]).astype(_I32)
            a0_ref[...] = a0
            af_ref[...] = jnp.where(a0 == 1, p0[:, 1], p0[:, 0] * p_sc[...][:, 0])

    return pl.pallas_call(
        body,
        grid=(_NBLK,),
        in_specs=[_blk_spec(bn, 1), _full_spec((G,)), _full_spec((G,)),
                  _full_spec((G, EMB)), _full_spec((G,)),
                  _full_spec((EMB, 1)), _full_spec((1,)),
                  _full_spec((EMB, 2)), _full_spec((2,))],
        out_specs=[_full_spec((G, 1)), _full_spec((G,)), _full_spec((G,))],
        out_shape=[jax.ShapeDtypeStruct((G, 1), _F32),
                   jax.ShapeDtypeStruct((G,), _F32),
                   jax.ShapeDtypeStruct((G,), _I32)],
        scratch_shapes=[pltpu.VMEM((G, 1), _F32)],
    )(np_col, starts, a1, xg, u0, Wv, bv, Wa0, ba0)


def kernel(x_raw, edge_index, batch_ind, u0, u1, params):
    n = x_raw.shape[0]
    g = u0.shape[0]
    unit = 128 * 8 * _C3BLK
    npad = ((n + unit - 1) // unit) * unit
    while npad % _NW or (npad // _NBLK) % 8 or (npad // 128 // _C3BLK) % 8:
        npad += unit

    xr = jnp.concatenate([x_raw, jnp.zeros((npad - n, 1), _F32)], axis=0)
    bi = jnp.concatenate(
        [batch_ind, jnp.full((npad - n,), g, _I32)], axis=0).reshape(npad, 1)
    esrc = edge_index[0]
    edst = edge_index[1]

    We, be = params["emb"]
    steps = params["steps"]
    x, y = _tc_emb(xr, We, be, steps[0]["mess"][0], steps[0]["mess"][1])
    bsrc, bdst, cnts = _sc_prepass(esrc, edst, npad)

    xg = jnp.zeros((g, EMB), _F32)
    m1 = None
    for s in range(3):
        p = steps[s]
        acc = _sc_segmax(y, bsrc, bdst, cnts, npad)
        if s < 2:
            next_p = steps[s + 1]["mess"]
            leaky_yn = True
        else:
            next_p = params["a1"]
            leaky_yn = False
        x, l, feat, y, m, m2 = _tc_b1(
            x, acc, bi, xg, p["agg"], p["gate"], p["feat"], next_p, leaky_yn)
        xg = _tc_b2(l, feat, bi, m, xg, p["tr"])
        m1 = m2

    e1, s1, cntg, starts = _tc_c2(y, bi, m1, n)
    np_col = _tc_c2b(e1, bi, s1)
    c128 = _tc_c3(np_col.reshape(npad // 128, 128))
    c_col = c128.reshape(npad, 1)
    a1 = _tc_c3b(c_col, bi, starts, cntg, u1)
    value, af, a0 = _tc_c4(np_col, starts, a1, xg, u0, params["v"], params["a0"])
    return (value, af, np_col[:n, 0], a0, a1)


# speedup vs baseline: 1.5190x; 1.5171x over previous
import functools

import jax
import jax.numpy as jnp
from jax import lax
from jax.experimental import pallas as pl
from jax.experimental.pallas import tpu as pltpu
from jax.experimental.pallas import tpu_sc as plsc

EMB = 32
G = 64
_NC = 2
_NS = 16
_NW = _NC * _NS
_CAP = 2048
_GC = 128
_NBLK = 32
_C3BLK = 10
_F32 = jnp.float32
_I32 = jnp.int32
_HI = lax.Precision.HIGHEST


def _leaky(x):
    return jnp.where(x >= 0, x, 0.01 * x)


def _full_spec(shape):
    return pl.BlockSpec(shape, lambda i: tuple(0 for _ in shape))


def _blk_spec(bn, minor):
    return pl.BlockSpec((bn, minor), lambda i: (i, 0))


def _sc_prepass(esrc, edst, npad):
    e = esrc.shape[0]
    echunk = e // _NW
    npn = npad // _NW
    nv = echunk // 16
    mesh = plsc.VectorSubcoreMesh(
        core_axis_name="c", subcore_axis_name="s",
        num_cores=_NC, num_subcores=_NS)

    @functools.partial(
        pl.kernel,
        out_type=[
            jax.ShapeDtypeStruct((_NW, _NW, _CAP), _I32),
            jax.ShapeDtypeStruct((_NW, _NW, _CAP), _I32),
            jax.ShapeDtypeStruct((_NW, _NW), _I32),
        ],
        mesh=mesh,
        scratch_types=[
            pltpu.VMEM((echunk,), _I32),
            pltpu.VMEM((echunk,), _I32),
            pltpu.VMEM((_CAP,), _I32),
            pltpu.VMEM((_CAP,), _I32),
            pltpu.VMEM((_NW,), _I32),
        ],
        compiler_params=pltpu.CompilerParams(needs_layout_passes=False),
    )
    def kern(esrc_h, edst_h, bsrc_h, bdst_h, cnt_h, src_v, dst_v, bs_v, bd_v, cnt_v):
        wid = lax.axis_index("s") * _NC + lax.axis_index("c")
        base = wid * echunk
        pltpu.sync_copy(esrc_h.at[pl.ds(base, echunk)], src_v)
        pltpu.sync_copy(edst_h.at[pl.ds(base, echunk)], dst_v)
        iota = lax.iota(_I32, 16)
        zeros16 = jnp.zeros((16,), _I32)
        dump16 = jnp.full((16,), npn, _I32)
        for o in range(_NW):
            lo = o * npn
            hi = lo + npn

            def vbody(k, cnt):
                idx = k * 16 + iota
                d = plsc.load_gather(dst_v, [idx])
                s = plsc.load_gather(src_v, [idx])
                m = (d >= lo) & (d < hi)
                plsc.store_compressed(bs_v.at[pl.ds(cnt, 16)], s, mask=m)
                plsc.store_compressed(bd_v.at[pl.ds(cnt, 16)], d - lo, mask=m)
                pc = plsc.all_reduce_population_count(m)[0]
                return jnp.minimum(cnt + pc, _CAP - 16)

            cnt = lax.fori_loop(0, nv, vbody, jnp.asarray(0, _I32))
            pend = ((cnt + 2 * _GC - 1) // (2 * _GC)) * (2 * _GC)
            for kk in range(2 * _GC // 16):
                pos = cnt + kk * 16 + iota
                mpad = pos < pend
                plsc.store_scatter(bs_v, [pos], zeros16, mask=mpad)
                plsc.store_scatter(bd_v, [pos], dump16, mask=mpad)
            pltpu.sync_copy(bs_v, bsrc_h.at[wid, o])
            pltpu.sync_copy(bd_v, bdst_h.at[wid, o])
            plsc.store_scatter(cnt_v, [jnp.full((16,), o, _I32)],
                               jnp.full((16,), cnt, _I32), mask=(iota == 0))
        pltpu.sync_copy(cnt_v, cnt_h.at[wid])

    return kern(esrc, edst)


def _sc_segmax(y, bsrc, bdst, cnts, npad):
    npn = npad // _NW
    mesh = plsc.VectorSubcoreMesh(
        core_axis_name="c", subcore_axis_name="s",
        num_cores=_NC, num_subcores=_NS)

    @functools.partial(
        pl.kernel,
        out_type=jax.ShapeDtypeStruct((npad, EMB), _F32),
        mesh=mesh,
        scratch_types=[
            pltpu.VMEM((npn, EMB), _F32),
            pltpu.VMEM((_CAP,), _I32),
            pltpu.VMEM((_CAP,), _I32),
            pltpu.VMEM((_NW, _NW), _I32),
            pltpu.VMEM((_GC, EMB), _F32),
            pltpu.SemaphoreType.DMA,
        ],
        compiler_params=pltpu.CompilerParams(
            needs_layout_passes=False, use_tc_tiling_on_sc=False),
    )
    def kern(y_h, bsrc_h, bdst_h, cnt_h, acc_h, acc_v, sb_v, db_v, cnt_v, msg_v, sem):
        wid = lax.axis_index("s") * _NC + lax.axis_index("c")
        o = wid
        iota = lax.iota(_I32, 16)
        neg = jnp.full((16,), -jnp.inf, _F32)

        def initb(r, _):
            rv = jnp.full((16,), r, _I32)
            plsc.store_scatter(acc_v, [rv, iota], neg)
            plsc.store_scatter(acc_v, [rv, iota + 16], neg)
            return 0

        lax.fori_loop(0, npn, initb, 0)
        pltpu.sync_copy(cnt_h, cnt_v)

        def wbody(w, _):
            wv = jnp.full((16,), w, _I32)
            ov = jnp.full((16,), o, _I32)
            cnt = plsc.load_gather(cnt_v, [wv, ov])[0]
            pltpu.sync_copy(bsrc_h.at[w, o], sb_v)
            pltpu.sync_copy(bdst_h.at[w, o], db_v)
            nch = (cnt + _GC - 1) // _GC

            def cbody(cc, _2):
                pltpu.async_copy(
                    y_h.at[sb_v.at[pl.ds(cc * _GC, _GC)]], msg_v, sem).wait()
                ne = jnp.minimum(cnt - cc * _GC, _GC)

                def gbody(k8, _3):
                    dlv = jnp.clip(
                        plsc.load_gather(db_v, [cc * _GC + k8 * 16 + iota]),
                        0, npn - 1)
                    for j2 in range(16):
                        dl = dlv[j2]
                        mvec = jnp.full((16,), k8 * 16 + j2 < ne)
                        dv = jnp.full((16,), dl, _I32)
                        jv = jnp.full((16,), k8 * 16 + j2, _I32)
                        r0 = plsc.load_gather(acc_v, [dv, iota])
                        r1 = plsc.load_gather(acc_v, [dv, iota + 16])
                        m0 = plsc.load_gather(msg_v, [jv, iota])
                        m1 = plsc.load_gather(msg_v, [jv, iota + 16])
                        plsc.store_scatter(acc_v, [dv, iota],
                                           jnp.maximum(r0, m0), mask=mvec)
                        plsc.store_scatter(acc_v, [dv, iota + 16],
                                           jnp.maximum(r1, m1), mask=mvec)
                    return 0

                lax.fori_loop(0, _GC // 16, gbody, 0)
                return 0

            lax.fori_loop(0, nch, cbody, 0)
            return 0

        lax.fori_loop(0, _NW, wbody, 0)
        pltpu.sync_copy(acc_v, acc_h.at[pl.ds(o * npn, npn)])

    return kern(y, bsrc, bdst, cnts)


def _tc_emb(xr, We, be, Wm, bm):
    npad = xr.shape[0]
    bn = npad // _NBLK

    def body(xr_ref, We_ref, be_ref, Wm_ref, bm_ref, x_ref, y_ref):
        x = _leaky(xr_ref[...] * We_ref[...] + be_ref[...])
        x_ref[...] = x
        y_ref[...] = _leaky(jnp.dot(x, Wm_ref[...], precision=_HI) + bm_ref[...])

    return pl.pallas_call(
        body,
        grid=(_NBLK,),
        in_specs=[_blk_spec(bn, 1), _full_spec((1, EMB)), _full_spec((EMB,)),
                  _full_spec((EMB, EMB)), _full_spec((EMB,))],
        out_specs=[_blk_spec(bn, EMB), _blk_spec(bn, EMB)],
        out_shape=[jax.ShapeDtypeStruct((npad, EMB), _F32),
                   jax.ShapeDtypeStruct((npad, EMB), _F32)],
    )(xr, We, be, Wm, bm)


def _tc_b1(x, acc, bi, xg, agg_p, gate_p, feat_p, next_p, leaky_yn):
    npad = x.shape[0]
    bn = npad // _NBLK
    Wa, ba = agg_p
    Wg, bg = gate_p
    Wf, bf = feat_p
    Wn, bnn = next_p
    kn = Wn.shape[1]

    def body(x_ref, acc_ref, bi_ref, xg_ref, Wa_ref, ba_ref, Wg_ref, bg_ref,
             Wf_ref, bf_ref, Wn_ref, bn_ref,
             xn_ref, l_ref, feat_ref, yn_ref, m_ref, m2_ref, m_sc, m2_sc):
        i = pl.program_id(0)

        @pl.when(i == 0)
        def _():
            m_sc[...] = jnp.full((G,), -jnp.inf, _F32)
            m2_sc[...] = jnp.full((G,), -jnp.inf, _F32)

        x_ = x_ref[...]
        a = acc_ref[...]
        agg = jnp.where(jnp.isneginf(a), 0.0, a)
        bi_ = bi_ref[...]
        oh = (bi_ == lax.broadcasted_iota(_I32, (1, G), 1)).astype(_F32)
        xgb = jnp.dot(oh, xg_ref[...], precision=_HI)
        z = jnp.concatenate([x_, xgb, agg], axis=1)
        xn = _leaky(jnp.dot(z, Wa_ref[...], precision=_HI) + ba_ref[...]) + x_
        l = jnp.dot(xn, Wg_ref[...], precision=_HI) + bg_ref[...]
        feat = _leaky(jnp.dot(xn, Wf_ref[...], precision=_HI) + bf_ref[...])
        yn = jnp.dot(xn, Wn_ref[...], precision=_HI) + bn_ref[...]
        if leaky_yn:
            yn = _leaky(yn)
        m_sc[...] = jnp.maximum(m_sc[...], jnp.max(
            jnp.where(oh > 0, l, -jnp.inf), axis=0))
        m2_sc[...] = jnp.maximum(m2_sc[...], jnp.max(
            jnp.where(oh > 0, yn[:, 0:1], -jnp.inf), axis=0))
        xn_ref[...] = xn
        l_ref[...] = l
        feat_ref[...] = feat
        yn_ref[...] = yn
        m_ref[...] = m_sc[...]
        m2_ref[...] = m2_sc[...]

    return pl.pallas_call(
        body,
        grid=(_NBLK,),
        in_specs=[_blk_spec(bn, EMB), _blk_spec(bn, EMB), _blk_spec(bn, 1),
                  _full_spec((G, EMB)),
                  _full_spec((3 * EMB, EMB)), _full_spec((EMB,)),
                  _full_spec((EMB, 1)), _full_spec((1,)),
                  _full_spec((EMB, EMB)), _full_spec((EMB,)),
                  _full_spec((EMB, kn)), _full_spec((kn,))],
        out_specs=[_blk_spec(bn, EMB), _blk_spec(bn, 1), _blk_spec(bn, EMB),
                   _blk_spec(bn, kn), _full_spec((G,)), _full_spec((G,))],
        out_shape=[jax.ShapeDtypeStruct((npad, EMB), _F32),
                   jax.ShapeDtypeStruct((npad, 1), _F32),
                   jax.ShapeDtypeStruct((npad, EMB), _F32),
                   jax.ShapeDtypeStruct((npad, kn), _F32),
                   jax.ShapeDtypeStruct((G,), _F32),
                   jax.ShapeDtypeStruct((G,), _F32)],
        scratch_shapes=[pltpu.VMEM((G,), _F32), pltpu.VMEM((G,), _F32)],
    )(x, acc, bi, xg, Wa, ba, Wg, bg, Wf, bf, Wn, bnn)


def _tc_b2(l, feat, bi, m, xg, tr_p):
    npad = l.shape[0]
    bn = npad // _NBLK
    Wt, bt = tr_p

    def body(l_ref, feat_ref, bi_ref, m_ref, xg_ref, Wt_ref, bt_ref,
             xgn_ref, s_sc, a_sc):
        i = pl.program_id(0)

        @pl.when(i == 0)
        def _():
            s_sc[...] = jnp.zeros((G, 1), _F32)
            a_sc[...] = jnp.zeros((G, EMB), _F32)

        bi_ = bi_ref[...]
        oh = (bi_ == lax.broadcasted_iota(_I32, (1, G), 1)).astype(_F32)
        m_ = m_ref[...]
        mf = jnp.where(jnp.isneginf(m_), 0.0, m_)
        mg = jnp.dot(oh, mf[:, None], precision=_HI)
        e = jnp.exp(l_ref[...] - mg)
        dn = (((0,), (0,)), ((), ()))
        s_sc[...] += lax.dot_general(oh, e, dn, precision=_HI)
        a_sc[...] += lax.dot_general(oh, e * feat_ref[...], dn, precision=_HI)

        @pl.when(i == _NBLK - 1)
        def _():
            xga = a_sc[...] / (s_sc[...] + 1e-16)
            xg_ = xg_ref[...]
            cat = jnp.concatenate([xga, xg_], axis=1)
            xgn_ref[...] = _leaky(
                jnp.dot(cat, Wt_ref[...], precision=_HI) + bt_ref[...]) + xg_

    return pl.pallas_call(
        body,
        grid=(_NBLK,),
        in_specs=[_blk_spec(bn, 1), _blk_spec(bn, EMB), _blk_spec(bn, 1),
                  _full_spec((G,)), _full_spec((G, EMB)),
                  _full_spec((2 * EMB, EMB)), _full_spec((EMB,))],
        out_specs=[_full_spec((G, EMB))],
        out_shape=[jax.ShapeDtypeStruct((G, EMB), _F32)],
        scratch_shapes=[pltpu.VMEM((G, 1), _F32), pltpu.VMEM((G, EMB), _F32)],
    )(l, feat, bi, m, xg, Wt, bt)[0]


def _tc_c2(l1, bi, m1, n_real):
    npad = l1.shape[0]
    bn = npad // _NBLK

    def body(l1_ref, bi_ref, m1_ref, e1_ref, s1_ref, cnt_ref, st_ref, s_sc, c_sc):
        i = pl.program_id(0)

        @pl.when(i == 0)
        def _():
            s_sc[...] = jnp.zeros((G, 1), _F32)
            c_sc[...] = jnp.zeros((G, 1), _F32)

        bi_ = bi_ref[...]
        oh = (bi_ == lax.broadcasted_iota(_I32, (1, G), 1)).astype(_F32)
        m_ = m1_ref[...]
        mf = jnp.where(jnp.isneginf(m_), 0.0, m_)
        mg = jnp.dot(oh, mf[:, None], precision=_HI)
        rowid = i * bn + lax.broadcasted_iota(_I32, (bn, 1), 0)
        e1 = jnp.where(rowid < n_real, jnp.exp(l1_ref[...] - mg), 0.0)
        dn = (((0,), (0,)), ((), ()))
        s_sc[...] += lax.dot_general(oh, e1, dn, precision=_HI)
        c_sc[...] += jnp.sum(oh, axis=0)[:, None]
        e1_ref[...] = e1

        @pl.when(i == _NBLK - 1)
        def _():
            s1_ref[...] = s_sc[...][:, 0]
            cnts = c_sc[...][:, 0]
            cnt_ref[...] = cnts.astype(_I32)
            rr = lax.broadcasted_iota(_I32, (G, G), 0)
            cc = lax.broadcasted_iota(_I32, (G, G), 1)
            lt = (cc < rr).astype(_F32)
            st_ref[...] = jnp.dot(lt, cnts[:, None], precision=_HI)[:, 0].astype(_I32)

    return pl.pallas_call(
        body,
        grid=(_NBLK,),
        in_specs=[_blk_spec(bn, 1), _blk_spec(bn, 1), _full_spec((G,))],
        out_specs=[_blk_spec(bn, 1), _full_spec((G,)), _full_spec((G,)),
                   _full_spec((G,))],
        out_shape=[jax.ShapeDtypeStruct((npad, 1), _F32),
                   jax.ShapeDtypeStruct((G,), _F32),
                   jax.ShapeDtypeStruct((G,), _I32),
                   jax.ShapeDtypeStruct((G,), _I32)],
        scratch_shapes=[pltpu.VMEM((G, 1), _F32), pltpu.VMEM((G, 1), _F32)],
    )(l1, bi, m1)


def _tc_c2b(e1, bi, s1):
    npad = e1.shape[0]
    bn = npad // _NBLK

    def body(e1_ref, bi_ref, s1_ref, np_ref):
        bi_ = bi_ref[...]
        oh = (bi_ == lax.broadcasted_iota(_I32, (1, G), 1)).astype(_F32)
        s1g = jnp.dot(oh, s1_ref[...][:, None], precision=_HI)
        np_ref[...] = e1_ref[...] / (s1g + 1e-16)

    return pl.pallas_call(
        body,
        grid=(_NBLK,),
        in_specs=[_blk_spec(bn, 1), _blk_spec(bn, 1), _full_spec((G,))],
        out_specs=[_blk_spec(bn, 1)],
        out_shape=[jax.ShapeDtypeStruct((npad, 1), _F32)],
    )(e1, bi, s1)[0]


def _tc_c3(np128):
    nrows = np128.shape[0]
    br = nrows // _C3BLK

    def body(np_ref, c_ref, carry_sc):
        i = pl.program_id(0)

        @pl.when(i == 0)
        def _():
            carry_sc[0] = 0.0

        v = np_ref[...]
        rr = lax.broadcasted_iota(_I32, (128, 128), 0)
        cc = lax.broadcasted_iota(_I32, (128, 128), 1)
        t = (rr <= cc).astype(_F32)
        rowcs = jnp.dot(v, t, precision=_HI)
        rowsum = rowcs[:, 127:128]
        r2 = lax.broadcasted_iota(_I32, (br, br), 0)
        c2 = lax.broadcasted_iota(_I32, (br, br), 1)
        lt = (c2 < r2).astype(_F32)
        rpref = jnp.dot(lt, rowsum, precision=_HI)
        carry = carry_sc[0]
        c_ref[...] = rowcs + rpref + carry
        carry_sc[0] = carry + (rpref[br - 1, 0] + rowsum[br - 1, 0])

    return pl.pallas_call(
        body,
        grid=(_C3BLK,),
        in_specs=[_blk_spec(br, 128)],
        out_specs=[_blk_spec(br, 128)],
        out_shape=[jax.ShapeDtypeStruct((nrows, 128), _F32)],
        scratch_shapes=[pltpu.SMEM((1,), _F32)],
    )(np128)[0]


def _tc_c3b(c_col, bi, starts, cnts, u1):
    npad = c_col.shape[0]
    bn = npad // _NBLK

    def body(c_ref, bi_ref, st_ref, cnt_ref, u1_ref, a1_ref, off_sc, k_sc):
        i = pl.program_id(0)

        @pl.when(i == 0)
        def _():
            off_sc[...] = jnp.zeros((G, 1), _F32)
            k_sc[...] = jnp.zeros((G, 1), _F32)

        bi_ = bi_ref[...]
        oh = (bi_ == lax.broadcasted_iota(_I32, (1, G), 1)).astype(_F32)
        c = c_ref[...]
        gpos = i * bn + lax.broadcasted_iota(_I32, (bn, 1), 0)
        st = st_ref[...]
        pick = (gpos == (st[None, :] - 1)).astype(_F32)
        dn = (((0,), (0,)), ((), ()))
        off_sc[...] += lax.dot_general(pick, c, dn, precision=_HI)
        offg = jnp.dot(oh, off_sc[...], precision=_HI)
        u1g = jnp.dot(oh, u1_ref[...][:, None], precision=_HI)
        kc = ((c - offg) < u1g).astype(_F32)
        k_sc[...] += lax.dot_general(oh, kc, dn, precision=_HI)

        @pl.when(i == _NBLK - 1)
        def _():
            k = k_sc[...][:, 0].astype(_I32)
            a1_ref[...] = jnp.clip(k, 0, jnp.maximum(cnt_ref[...] - 1, 0))

    return pl.pallas_call(
        body,
        grid=(_NBLK,),
        in_specs=[_blk_spec(bn, 1), _blk_spec(bn, 1), _full_spec((G,)),
                  _full_spec((G,)), _full_spec((G,))],
        out_specs=[_full_spec((G,))],
        out_shape=[jax.ShapeDtypeStruct((G,), _I32)],
        scratch_shapes=[pltpu.VMEM((G, 1), _F32), pltpu.VMEM((G, 1), _F32)],
    )(c_col, bi, starts, cnts, u1)[0]


def _tc_c4(np_col, starts, a1, xg, u0, v_p, a0_p):
    npad = np_col.shape[0]
    bn = npad // _NBLK
    Wv, bv = v_p
    Wa0, ba0 = a0_p

    def body(np_ref, st_ref, a1_ref, xg_ref, u0_ref, Wv_ref, bv_ref,
             Wa0_ref, ba0_ref, val_ref, af_ref, a0_ref, p_sc):
        i = pl.program_id(0)

        @pl.when(i == 0)
        def _():
            p_sc[...] = jnp.zeros((G, 1), _F32)

        gpos = i * bn + lax.broadcasted_iota(_I32, (bn, 1), 0)
        t = st_ref[...] + a1_ref[...]
        pick = (gpos == t[None, :]).astype(_F32)
        dn = (((0,), (0,)), ((), ()))
        p_sc[...] += lax.dot_general(pick, np_ref[...], dn, precision=_HI)

        @pl.when(i == _NBLK - 1)
        def _():
            xg_ = xg_ref[...]
            val_ref[...] = jnp.dot(xg_, Wv_ref[...], precision=_HI) + bv_ref[...]
            tt = jnp.dot(xg_, Wa0_ref[...], precision=_HI) + ba0_ref[...]
            mm = jnp.max(tt, axis=1, keepdims=True)
            ex = jnp.exp(tt - mm)
            p0 = ex / jnp.sum(ex, axis=1, keepdims=True)
            a0 = (u0_ref[...] >= p0[:, 0]).astype(_I32)
            a0_ref[...] = a0
            af_ref[...] = jnp.where(a0 == 1, p0[:, 1], p0[:, 0] * p_sc[...][:, 0])

    return pl.pallas_call(
        body,
        grid=(_NBLK,),
        in_specs=[_blk_spec(bn, 1), _full_spec((G,)), _full_spec((G,)),
                  _full_spec((G, EMB)), _full_spec((G,)),
                  _full_spec((EMB, 1)), _full_spec((1,)),
                  _full_spec((EMB, 2)), _full_spec((2,))],
        out_specs=[_full_spec((G, 1)), _full_spec((G,)), _full_spec((G,))],
        out_shape=[jax.ShapeDtypeStruct((G, 1), _F32),
                   jax.ShapeDtypeStruct((G,), _F32),
                   jax.ShapeDtypeStruct((G,), _I32)],
        scratch_shapes=[pltpu.VMEM((G, 1), _F32)],
    )(np_col, starts, a1, xg, u0, Wv, bv, Wa0, ba0)


def kernel(x_raw, edge_index, batch_ind, u0, u1, params):
    n = x_raw.shape[0]
    g = u0.shape[0]
    unit = 128 * 8 * _C3BLK
    npad = ((n + unit - 1) // unit) * unit
    while npad % _NW or (npad // _NBLK) % 8 or (npad // 128 // _C3BLK) % 8:
        npad += unit

    xr = jnp.concatenate([x_raw, jnp.zeros((npad - n, 1), _F32)], axis=0)
    bi = jnp.concatenate(
        [batch_ind, jnp.full((npad - n,), g, _I32)], axis=0).reshape(npad, 1)
    esrc = edge_index[0]
    edst = edge_index[1]

    We, be = params["emb"]
    steps = params["steps"]
    x, y = _tc_emb(xr, We, be, steps[0]["mess"][0], steps[0]["mess"][1])
    bsrc, bdst, cnts = _sc_prepass(esrc, edst, npad)

    xg = jnp.zeros((g, EMB), _F32)
    m1 = None
    for s in range(3):
        p = steps[s]
        acc = _sc_segmax(y, bsrc, bdst, cnts, npad)
        if s < 2:
            next_p = steps[s + 1]["mess"]
            leaky_yn = True
        else:
            next_p = params["a1"]
            leaky_yn = False
        x, l, feat, y, m, m2 = _tc_b1(
            x, acc, bi, xg, p["agg"], p["gate"], p["feat"], next_p, leaky_yn)
        xg = _tc_b2(l, feat, bi, m, xg, p["tr"])
        m1 = m2

    e1, s1, cntg, starts = _tc_c2(y, bi, m1, n)
    np_col = _tc_c2b(e1, bi, s1)
    c128 = _tc_c3(np_col.reshape(npad // 128, 128))
    c_col = c128.reshape(npad, 1)
    a1 = _tc_c3b(c_col, bi, starts, cntg, u1)
    value, af, a0 = _tc_c4(np_col, starts, a1, xg, u0, params["v"], params["a0"])
    return (value, af, np_col[:n, 0], a0, a1)
